# Initial kernel scaffold; baseline (speedup 1.0000x reference)
import jax
import jax.numpy as jnp
from jax.experimental import pallas as pl


def kernel(x, edge_index, W1l, b1l, W1r, b1r, att1, bias1, W2l, b2l, W2r, b2r, att2, bias2):
    raise NotImplementedError("write your pallas kernel here")


# speedup vs baseline: 18.1677x; 18.1677x over previous
import functools

import jax
import jax.numpy as jnp
from jax import lax
from jax.experimental import pallas as pl
from jax.experimental.pallas import tpu as pltpu
from jax.experimental.pallas import tpu_sc as plsc

NC = 2
NS = 16
NW = NC * NS
CHUNK = 128
MASK_NEG = -1e30


def _mesh():
    return plsc.VectorSubcoreMesh(core_axis_name="c", subcore_axis_name="s")


def _sc_gather2(table_a, table_b, idx_a, idx_b, cpw):
    _, d = table_a.shape
    e_pad = idx_a.shape[0]

    @functools.partial(
        pl.kernel,
        mesh=_mesh(),
        out_type=(jax.ShapeDtypeStruct((e_pad, d), jnp.float32),
                  jax.ShapeDtypeStruct((e_pad, d), jnp.float32)),
        scratch_types=[pltpu.VMEM((CHUNK,), jnp.int32),
                       pltpu.VMEM((CHUNK,), jnp.int32),
                       pltpu.VMEM((CHUNK, d), jnp.float32),
                       pltpu.VMEM((CHUNK, d), jnp.float32),
                       pltpu.SemaphoreType.DMA,
                       pltpu.SemaphoreType.DMA],
    )
    def k(ta, tb, ia, ib, oa, ob, iva, ivb, ra, rb, sa, sb):
        wid = lax.axis_index("s") * NC + lax.axis_index("c")

        def step(j, carry):
            base = (wid * cpw + j) * CHUNK
            pltpu.sync_copy(ia.at[pl.ds(base, CHUNK)], iva)
            pltpu.sync_copy(ib.at[pl.ds(base, CHUNK)], ivb)
            ca = pltpu.async_copy(ta.at[iva], ra, sa)
            cb = pltpu.async_copy(tb.at[ivb], rb, sb)
            ca.wait()
            cb.wait()
            pltpu.sync_copy(ra, oa.at[pl.ds(base, CHUNK)])
            pltpu.sync_copy(rb, ob.at[pl.ds(base, CHUNK)])
            return carry

        lax.fori_loop(0, cpw, step, 0)

    return k(table_a, table_b, idx_a, idx_b)


def _sc_scatter_add(vals, idx, zeros, cpw):
    e_pad, w = vals.shape
    n = zeros.shape[0]
    assert n % (NS * 8) == 0
    rps = n // NS

    @functools.partial(
        pl.kernel,
        mesh=_mesh(),
        out_type=jax.ShapeDtypeStruct((NC, n, w), jnp.float32),
        scratch_types=[pltpu.VMEM((CHUNK,), jnp.int32),
                       pltpu.VMEM((CHUNK, w), jnp.float32),
                       pltpu.VMEM_SHARED((n, w), jnp.float32)],
    )
    def k(vals_h, idx_h, zeros_h, out_h, iv, vv, shared):
        cid = lax.axis_index("c")
        sid = lax.axis_index("s")
        wid = sid * NC + cid
        pltpu.sync_copy(zeros_h.at[pl.ds(sid * rps, rps)],
                        shared.at[pl.ds(sid * rps, rps)])
        plsc.subcore_barrier()

        def step(j, carry):
            base = (wid * cpw + j) * CHUNK
            pltpu.sync_copy(idx_h.at[pl.ds(base, CHUNK)], iv)
            pltpu.sync_copy(vals_h.at[pl.ds(base, CHUNK)], vv)
            pltpu.sync_copy(vv, shared.at[iv], add=True)
            return carry

        lax.fori_loop(0, cpw, step, 0)
        plsc.subcore_barrier()
        pltpu.sync_copy(shared.at[pl.ds(sid * rps, rps)],
                        out_h.at[cid, pl.ds(sid * rps, rps)])

    return k(vals, idx, zeros)


def _lin_body(x_ref, wl_ref, bl_ref, wr_ref, br_ref, ol_ref, or_ref):
    xv = x_ref[...]
    ol_ref[...] = jnp.dot(xv, wl_ref[...],
                          preferred_element_type=jnp.float32) + bl_ref[...]
    or_ref[...] = jnp.dot(xv, wr_ref[...],
                          preferred_element_type=jnp.float32) + br_ref[...]


def _tc_lin(x, wl, bl, wr, br):
    n, d = x.shape
    hc = wl.shape[1]
    blk = n // 10
    return pl.pallas_call(
        _lin_body,
        grid=(n // blk,),
        in_specs=[pl.BlockSpec((blk, d), lambda i: (i, 0)),
                  pl.BlockSpec((d, hc), lambda i: (0, 0)),
                  pl.BlockSpec((1, hc), lambda i: (0, 0)),
                  pl.BlockSpec((d, hc), lambda i: (0, 0)),
                  pl.BlockSpec((1, hc), lambda i: (0, 0))],
        out_specs=(pl.BlockSpec((blk, hc), lambda i: (i, 0)),
                   pl.BlockSpec((blk, hc), lambda i: (i, 0))),
        out_shape=(jax.ShapeDtypeStruct((n, hc), jnp.float32),
                   jax.ShapeDtypeStruct((n, hc), jnp.float32)),
    )(x, wl, bl.reshape(1, hc), wr, br.reshape(1, hc))


def _alpha_body(gj_ref, gi_ref, s_ref, o_ref, *, blk, e_act):
    z = gj_ref[...] + gi_ref[...]
    a = jnp.maximum(z, 0.2 * z)
    al = jnp.dot(a, s_ref[...], preferred_element_type=jnp.float32)
    row = pl.program_id(0) * blk + lax.broadcasted_iota(jnp.int32, al.shape, 0)
    o_ref[...] = jnp.where(row < e_act, al, MASK_NEG)


def _tc_alpha(gj, gi, s, e_act, blk=2048):
    e_pad, d = gj.shape
    h8 = s.shape[1]
    return pl.pallas_call(
        functools.partial(_alpha_body, blk=blk, e_act=e_act),
        grid=(e_pad // blk,),
        in_specs=[pl.BlockSpec((blk, d), lambda i: (i, 0)),
                  pl.BlockSpec((blk, d), lambda i: (i, 0)),
                  pl.BlockSpec((d, h8), lambda i: (0, 0))],
        out_specs=pl.BlockSpec((blk, h8), lambda i: (i, 0)),
        out_shape=jax.ShapeDtypeStruct((e_pad, h8), jnp.float32),
    )(gj, gi, s)


def _gmax_body(a_ref, o_ref):
    @pl.when(pl.program_id(0) == 0)
    def _():
        o_ref[...] = jnp.full_like(o_ref[...], -jnp.inf)

    o_ref[...] = jnp.maximum(o_ref[...],
                             jnp.max(a_ref[...], axis=0, keepdims=True))


def _tc_gmax(alpha, blk=8192):
    e_pad, h8 = alpha.shape
    return pl.pallas_call(
        _gmax_body,
        grid=(e_pad // blk,),
        in_specs=[pl.BlockSpec((blk, h8), lambda i: (i, 0))],
        out_specs=pl.BlockSpec((1, h8), lambda i: (0, 0)),
        out_shape=jax.ShapeDtypeStruct((1, h8), jnp.float32),
    )(alpha)


def _ex_body(a_ref, g_ref, o_ref):
    o_ref[...] = jnp.exp(a_ref[...] - g_ref[...])


def _tc_ex(alpha, gmax, blk=8192):
    e_pad, h8 = alpha.shape
    return pl.pallas_call(
        _ex_body,
        grid=(e_pad // blk,),
        in_specs=[pl.BlockSpec((blk, h8), lambda i: (i, 0)),
                  pl.BlockSpec((1, h8), lambda i: (0, 0))],
        out_specs=pl.BlockSpec((blk, h8), lambda i: (i, 0)),
        out_shape=jax.ShapeDtypeStruct((e_pad, h8), jnp.float32),
    )(alpha, gmax)


def _msg_body(gj_ref, ex_ref, x_ref, o_ref):
    o_ref[...] = gj_ref[...] * jnp.dot(ex_ref[...], x_ref[...],
                                       preferred_element_type=jnp.float32)


def _tc_msg(gj, ex, x, blk=2048):
    e_pad, d = gj.shape
    h8 = ex.shape[1]
    return pl.pallas_call(
        _msg_body,
        grid=(e_pad // blk,),
        in_specs=[pl.BlockSpec((blk, d), lambda i: (i, 0)),
                  pl.BlockSpec((blk, h8), lambda i: (i, 0)),
                  pl.BlockSpec((h8, d), lambda i: (0, 0))],
        out_specs=pl.BlockSpec((blk, d), lambda i: (i, 0)),
        out_shape=jax.ShapeDtypeStruct((e_pad, d), jnp.float32),
    )(gj, ex, x)


def _out_body(o2_ref, d2_ref, x_ref, b_ref, o_ref, *, elu):
    inv = 1.0 / (d2_ref[0] + d2_ref[1] + 1e-16)
    ivx = jnp.dot(inv, x_ref[...], preferred_element_type=jnp.float32)
    z = (o2_ref[0] + o2_ref[1]) * ivx + b_ref[...]
    if elu:
        z = jnp.where(z > 0, z, jnp.exp(jnp.minimum(z, 0.0)) - 1.0)
    o_ref[...] = z


def _tc_out(opart, dpart, x, bias, elu, n):
    d = opart.shape[2]
    h8 = dpart.shape[2]
    blk = n // 10
    return pl.pallas_call(
        functools.partial(_out_body, elu=elu),
        grid=(n // blk,),
        in_specs=[pl.BlockSpec((2, blk, d), lambda i: (0, i, 0)),
                  pl.BlockSpec((2, blk, h8), lambda i: (0, i, 0)),
                  pl.BlockSpec((h8, d), lambda i: (0, 0)),
                  pl.BlockSpec((1, d), lambda i: (0, 0))],
        out_specs=pl.BlockSpec((blk, d), lambda i: (i, 0)),
        out_shape=jax.ShapeDtypeStruct((n, d), jnp.float32),
    )(opart, dpart, x, bias.reshape(1, d))


def _layer(x, src, dst, wl, bl, wr, br, s, xmat, bias, zeros8, zeros128,
           e_act, cpw, elu):
    xl, xr = _tc_lin(x, wl, bl, wr, br)
    gj, gi = _sc_gather2(xl, xr, src, dst, cpw)
    alpha = _tc_alpha(gj, gi, s, e_act)
    gmax = _tc_gmax(alpha)
    ex = _tc_ex(alpha, gmax)
    dpart = _sc_scatter_add(ex, dst, zeros8, cpw)
    msg = _tc_msg(gj, ex, xmat)
    opart = _sc_scatter_add(msg, dst, zeros128, cpw)
    return _tc_out(opart, dpart, xmat, bias, elu, x.shape[0])


def kernel(x, edge_index, W1l, b1l, W1r, b1r, att1, bias1,
           W2l, b2l, W2r, b2r, att2, bias2):
    n, d = x.shape
    e_in = edge_index.shape[1]
    e_act = e_in + n
    grain = NW * CHUNK
    e_pad = ((e_act + grain - 1) // grain) * grain
    cpw = e_pad // grain

    loops = jnp.arange(n, dtype=edge_index.dtype)
    padi = jnp.zeros((e_pad - e_act,), edge_index.dtype)
    src = jnp.concatenate([edge_index[0], loops, padi])
    dst = jnp.concatenate([edge_index[1], loops, padi])

    eye8 = jnp.eye(8, dtype=jnp.float32)
    s1 = (eye8[:, None, :] * att1[:, :, None]).reshape(d, 8)
    x1 = jnp.repeat(eye8, 16, axis=1)
    s2 = jnp.zeros((d, 8), jnp.float32).at[:, 0].set(att2[0])
    x2 = jnp.zeros((8, d), jnp.float32).at[0, :].set(1.0)

    n_pad = ((n + NS * 8 - 1) // (NS * 8)) * (NS * 8)
    zeros8 = jnp.zeros((n_pad, 8), jnp.float32)
    zeros128 = jnp.zeros((n_pad, d), jnp.float32)

    h = _layer(x, src, dst, W1l, b1l, W1r, b1r, s1, x1, bias1,
               zeros8, zeros128, e_act, cpw, elu=True)
    return _layer(h, src, dst, W2l, b2l, W2r, b2r, s2, x2, bias2,
                  zeros8, zeros128, e_act, cpw, elu=False)


# speedup vs baseline: 20.0394x; 1.1030x over previous
import functools

import jax
import jax.numpy as jnp
from jax import lax
from jax.experimental import pallas as pl
from jax.experimental.pallas import tpu as pltpu
from jax.experimental.pallas import tpu_sc as plsc

NC = 2
NS = 16
NW = NC * NS
CHUNK = 128
MASK_NEG = -1e30


def _mesh():
    return plsc.VectorSubcoreMesh(core_axis_name="c", subcore_axis_name="s")


def _sc_gather2(table_a, table_b, idx_a, idx_b, cpw):
    _, d = table_a.shape
    e_pad = idx_a.shape[0]

    @functools.partial(
        pl.kernel,
        mesh=_mesh(),
        out_type=(jax.ShapeDtypeStruct((e_pad, d), jnp.float32),
                  jax.ShapeDtypeStruct((e_pad, d), jnp.float32)),
        scratch_types=[pltpu.VMEM((CHUNK,), jnp.int32),
                       pltpu.VMEM((CHUNK,), jnp.int32),
                       pltpu.VMEM((CHUNK,), jnp.int32),
                       pltpu.VMEM((CHUNK,), jnp.int32),
                       pltpu.VMEM((CHUNK, d), jnp.float32),
                       pltpu.VMEM((CHUNK, d), jnp.float32),
                       pltpu.VMEM((CHUNK, d), jnp.float32),
                       pltpu.VMEM((CHUNK, d), jnp.float32),
                       pltpu.SemaphoreType.DMA,
                       pltpu.SemaphoreType.DMA,
                       pltpu.SemaphoreType.DMA,
                       pltpu.SemaphoreType.DMA,
                       pltpu.SemaphoreType.DMA,
                       pltpu.SemaphoreType.DMA,
                       pltpu.SemaphoreType.DMA,
                       pltpu.SemaphoreType.DMA],
    )
    def k(ta, tb, ia, ib, oa, ob,
          iva0, ivb0, iva1, ivb1, ra0, rb0, ra1, rb1,
          sa0, sb0, sa1, sb1, swa0, swb0, swa1, swb1):
        wid = lax.axis_index("s") * NC + lax.axis_index("c")
        bufs = ((iva0, ivb0, ra0, rb0, sa0, sb0, swa0, swb0),
                (iva1, ivb1, ra1, rb1, sa1, sb1, swa1, swb1))

        def fire(j, b):
            iva, ivb, ra, rb, sa, sb, _, _ = bufs[b]
            base = (wid * cpw + j) * CHUNK
            pltpu.sync_copy(ia.at[pl.ds(base, CHUNK)], iva)
            pltpu.sync_copy(ib.at[pl.ds(base, CHUNK)], ivb)
            return (pltpu.async_copy(ta.at[iva], ra, sa),
                    pltpu.async_copy(tb.at[ivb], rb, sb))

        def wb(j, b):
            iva, ivb, ra, rb, _, _, swa, swb = bufs[b]
            base = (wid * cpw + j) * CHUNK
            return (pltpu.async_copy(ra, oa.at[pl.ds(base, CHUNK)], swa),
                    pltpu.async_copy(rb, ob.at[pl.ds(base, CHUNK)], swb))

        gh = [None, None]
        wh = [None, None]
        gh[0] = fire(0, 0)
        for j in range(cpw):
            b = j % 2
            nb = 1 - b
            ca, cb = gh[b]
            ca.wait()
            cb.wait()
            if j + 1 < cpw:
                if wh[nb] is not None:
                    wh[nb][0].wait()
                    wh[nb][1].wait()
                    wh[nb] = None
                gh[nb] = fire(j + 1, nb)
            wh[b] = wb(j, b)
        for h in wh:
            if h is not None:
                h[0].wait()
                h[1].wait()

    return k(table_a, table_b, idx_a, idx_b)


def _sc_scatter_add(vals, idx, zeros, cpw):
    e_pad, w = vals.shape
    n = zeros.shape[0]
    assert n % (NS * 8) == 0
    rps = n // NS

    @functools.partial(
        pl.kernel,
        mesh=_mesh(),
        out_type=jax.ShapeDtypeStruct((NC, n, w), jnp.float32),
        scratch_types=[pltpu.VMEM((CHUNK,), jnp.int32),
                       pltpu.VMEM((CHUNK, w), jnp.float32),
                       pltpu.VMEM_SHARED((n, w), jnp.float32)],
    )
    def k(vals_h, idx_h, zeros_h, out_h, iv, vv, shared):
        cid = lax.axis_index("c")
        sid = lax.axis_index("s")
        wid = sid * NC + cid
        pltpu.sync_copy(zeros_h.at[pl.ds(sid * rps, rps)],
                        shared.at[pl.ds(sid * rps, rps)])
        plsc.subcore_barrier()

        def step(j, carry):
            base = (wid * cpw + j) * CHUNK
            pltpu.sync_copy(idx_h.at[pl.ds(base, CHUNK)], iv)
            pltpu.sync_copy(vals_h.at[pl.ds(base, CHUNK)], vv)
            pltpu.sync_copy(vv, shared.at[iv], add=True)
            return carry

        lax.fori_loop(0, cpw, step, 0)
        plsc.subcore_barrier()
        pltpu.sync_copy(shared.at[pl.ds(sid * rps, rps)],
                        out_h.at[cid, pl.ds(sid * rps, rps)])

    return k(vals, idx, zeros)


def _lin_body(x_ref, wl_ref, bl_ref, wr_ref, br_ref, ol_ref, or_ref):
    xv = x_ref[...]
    ol_ref[...] = jnp.dot(xv, wl_ref[...],
                          preferred_element_type=jnp.float32) + bl_ref[...]
    or_ref[...] = jnp.dot(xv, wr_ref[...],
                          preferred_element_type=jnp.float32) + br_ref[...]


def _tc_lin(x, wl, bl, wr, br):
    n, d = x.shape
    hc = wl.shape[1]
    blk = n // 10
    return pl.pallas_call(
        _lin_body,
        grid=(n // blk,),
        in_specs=[pl.BlockSpec((blk, d), lambda i: (i, 0)),
                  pl.BlockSpec((d, hc), lambda i: (0, 0)),
                  pl.BlockSpec((1, hc), lambda i: (0, 0)),
                  pl.BlockSpec((d, hc), lambda i: (0, 0)),
                  pl.BlockSpec((1, hc), lambda i: (0, 0))],
        out_specs=(pl.BlockSpec((blk, hc), lambda i: (i, 0)),
                   pl.BlockSpec((blk, hc), lambda i: (i, 0))),
        out_shape=(jax.ShapeDtypeStruct((n, hc), jnp.float32),
                   jax.ShapeDtypeStruct((n, hc), jnp.float32)),
    )(x, wl, bl.reshape(1, hc), wr, br.reshape(1, hc))


def _alpha_body(gj_ref, gi_ref, s_ref, o_ref, *, blk, e_act):
    z = gj_ref[...] + gi_ref[...]
    a = jnp.maximum(z, 0.2 * z)
    al = jnp.dot(a, s_ref[...], preferred_element_type=jnp.float32)
    row = pl.program_id(0) * blk + lax.broadcasted_iota(jnp.int32, al.shape, 0)
    o_ref[...] = jnp.where(row < e_act, al, MASK_NEG)


def _tc_alpha(gj, gi, s, e_act, blk=2048):
    e_pad, d = gj.shape
    h8 = s.shape[1]
    return pl.pallas_call(
        functools.partial(_alpha_body, blk=blk, e_act=e_act),
        grid=(e_pad // blk,),
        in_specs=[pl.BlockSpec((blk, d), lambda i: (i, 0)),
                  pl.BlockSpec((blk, d), lambda i: (i, 0)),
                  pl.BlockSpec((d, h8), lambda i: (0, 0))],
        out_specs=pl.BlockSpec((blk, h8), lambda i: (i, 0)),
        out_shape=jax.ShapeDtypeStruct((e_pad, h8), jnp.float32),
    )(gj, gi, s)


def _gmax_body(a_ref, o_ref):
    @pl.when(pl.program_id(0) == 0)
    def _():
        o_ref[...] = jnp.full_like(o_ref[...], -jnp.inf)

    o_ref[...] = jnp.maximum(o_ref[...],
                             jnp.max(a_ref[...], axis=0, keepdims=True))


def _tc_gmax(alpha, blk=8192):
    e_pad, h8 = alpha.shape
    return pl.pallas_call(
        _gmax_body,
        grid=(e_pad // blk,),
        in_specs=[pl.BlockSpec((blk, h8), lambda i: (i, 0))],
        out_specs=pl.BlockSpec((1, h8), lambda i: (0, 0)),
        out_shape=jax.ShapeDtypeStruct((1, h8), jnp.float32),
    )(alpha)


def _exmsg_body(a_ref, g_ref, gj_ref, x_ref, om_ref, oe_ref):
    ex = jnp.exp(a_ref[...] - g_ref[...])
    oe_ref[...] = ex
    om_ref[...] = gj_ref[...] * jnp.dot(ex, x_ref[...],
                                        preferred_element_type=jnp.float32)


def _tc_exmsg(alpha, gmax, gj, x, blk=2048):
    e_pad, d = gj.shape
    h8 = alpha.shape[1]
    return pl.pallas_call(
        _exmsg_body,
        grid=(e_pad // blk,),
        in_specs=[pl.BlockSpec((blk, h8), lambda i: (i, 0)),
                  pl.BlockSpec((1, h8), lambda i: (0, 0)),
                  pl.BlockSpec((blk, d), lambda i: (i, 0)),
                  pl.BlockSpec((h8, d), lambda i: (0, 0))],
        out_specs=(pl.BlockSpec((blk, d), lambda i: (i, 0)),
                   pl.BlockSpec((blk, h8), lambda i: (i, 0))),
        out_shape=(jax.ShapeDtypeStruct((e_pad, d), jnp.float32),
                   jax.ShapeDtypeStruct((e_pad, h8), jnp.float32)),
    )(alpha, gmax, gj, x)


def _out_body(o2_ref, d2_ref, x_ref, b_ref, o_ref, *, elu):
    inv = 1.0 / (d2_ref[0] + d2_ref[1] + 1e-16)
    ivx = jnp.dot(inv, x_ref[...], preferred_element_type=jnp.float32)
    z = (o2_ref[0] + o2_ref[1]) * ivx + b_ref[...]
    if elu:
        z = jnp.where(z > 0, z, jnp.exp(jnp.minimum(z, 0.0)) - 1.0)
    o_ref[...] = z


def _tc_out(opart, dpart, x, bias, elu, n):
    d = opart.shape[2]
    h8 = dpart.shape[2]
    blk = n // 10
    return pl.pallas_call(
        functools.partial(_out_body, elu=elu),
        grid=(n // blk,),
        in_specs=[pl.BlockSpec((2, blk, d), lambda i: (0, i, 0)),
                  pl.BlockSpec((2, blk, h8), lambda i: (0, i, 0)),
                  pl.BlockSpec((h8, d), lambda i: (0, 0)),
                  pl.BlockSpec((1, d), lambda i: (0, 0))],
        out_specs=pl.BlockSpec((blk, d), lambda i: (i, 0)),
        out_shape=jax.ShapeDtypeStruct((n, d), jnp.float32),
    )(opart, dpart, x, bias.reshape(1, d))


def _layer(x, src, dst, wl, bl, wr, br, s, xmat, bias, zeros8, zeros128,
           e_act, cpw, elu):
    xl, xr = _tc_lin(x, wl, bl, wr, br)
    gj, gi = _sc_gather2(xl, xr, src, dst, cpw)
    alpha = _tc_alpha(gj, gi, s, e_act)
    gmax = _tc_gmax(alpha)
    msg, ex = _tc_exmsg(alpha, gmax, gj, xmat)
    dpart = _sc_scatter_add(ex, dst, zeros8, cpw)
    opart = _sc_scatter_add(msg, dst, zeros128, cpw)
    return _tc_out(opart, dpart, xmat, bias, elu, x.shape[0])


def kernel(x, edge_index, W1l, b1l, W1r, b1r, att1, bias1,
           W2l, b2l, W2r, b2r, att2, bias2):
    n, d = x.shape
    e_in = edge_index.shape[1]
    e_act = e_in + n
    grain = NW * CHUNK
    e_pad = ((e_act + grain - 1) // grain) * grain
    cpw = e_pad // grain

    loops = jnp.arange(n, dtype=edge_index.dtype)
    padi = jnp.zeros((e_pad - e_act,), edge_index.dtype)
    src = jnp.concatenate([edge_index[0], loops, padi])
    dst = jnp.concatenate([edge_index[1], loops, padi])

    eye8 = jnp.eye(8, dtype=jnp.float32)
    s1 = (eye8[:, None, :] * att1[:, :, None]).reshape(d, 8)
    x1 = jnp.repeat(eye8, 16, axis=1)
    s2 = jnp.zeros((d, 8), jnp.float32).at[:, 0].set(att2[0])
    x2 = jnp.zeros((8, d), jnp.float32).at[0, :].set(1.0)

    n_pad = ((n + NS * 8 - 1) // (NS * 8)) * (NS * 8)
    zeros8 = jnp.zeros((n_pad, 8), jnp.float32)
    zeros128 = jnp.zeros((n_pad, d), jnp.float32)

    h = _layer(x, src, dst, W1l, b1l, W1r, b1r, s1, x1, bias1,
               zeros8, zeros128, e_act, cpw, elu=True)
    return _layer(h, src, dst, W2l, b2l, W2r, b2r, s2, x2, bias2,
                  zeros8, zeros128, e_act, cpw, elu=False)
